# baseline (speedup 1.0000x reference)
import functools

import jax
import jax.numpy as jnp
from jax import lax
from jax.experimental import pallas as pl
from jax.experimental.pallas import tpu as pltpu
from jax.experimental.pallas import tpu_sc as plsc

MAX_LEN = 200
N_FILTERS = 128
BATCH = 4096
SEQ = 200

_B = BATCH * SEQ
_NC = 2
_NS = 16
_NW = _NC * _NS
_F_SC_NUM, _F_SC_DEN = 1, 2
_B1 = (_B * _F_SC_NUM // _F_SC_DEN // (_NW * 400)) * (_NW * 400)
_B2 = _B - _B1
_PER_W = _B1 // _NW
_R = 400
_CHUNKS = _PER_W // _R

_TB = 512
_NB = _B2 // _TB


def _build_gather():
    mesh = plsc.VectorSubcoreMesh(core_axis_name="c", subcore_axis_name="s")

    @functools.partial(
        pl.kernel,
        mesh=mesh,
        out_type=jax.ShapeDtypeStruct((_B1, N_FILTERS), jnp.float32),
        scratch_types=[
            pltpu.VMEM_SHARED((MAX_LEN, N_FILTERS), jnp.float32),
            pltpu.VMEM((_R,), jnp.int32),
            pltpu.VMEM((_R,), jnp.int32),
            pltpu.VMEM((_R, N_FILTERS), jnp.float32),
            pltpu.VMEM((_R, N_FILTERS), jnp.float32),
            pltpu.SemaphoreType.DMA,
            pltpu.SemaphoreType.DMA,
            pltpu.SemaphoreType.DMA,
            pltpu.SemaphoreType.DMA,
            pltpu.SemaphoreType.DMA,
            pltpu.SemaphoreType.DMA,
        ],
    )
    def gather_kernel(table_hbm, idx_hbm, out_hbm,
                      table_sh, idx0, idx1, rows0, rows1,
                      si0, si1, sg0, sg1, so0, so1):
        wid = lax.axis_index("s") * _NC + lax.axis_index("c")
        base0 = wid * _PER_W
        idx_b = (idx0, idx1)
        rows_b = (rows0, rows1)
        si_b = (si0, si1)
        sg_b = (sg0, sg1)
        so_b = (so0, so1)

        def idx_slice(i):
            return idx_hbm.at[pl.ds(base0 + i * _R, _R)]

        def out_slice(i):
            return out_hbm.at[pl.ds(base0 + i * _R, _R)]

        @pl.when(lax.axis_index("s") == 0)
        def _():
            pltpu.sync_copy(table_hbm, table_sh)

        plsc.subcore_barrier()

        pltpu.async_copy(idx_slice(0), idx0, si0)
        pltpu.async_copy(idx_slice(1), idx1, si1)
        pltpu.make_async_copy(idx_slice(0), idx0, si0).wait()
        pltpu.async_copy(table_sh.at[idx0], rows0, sg0)

        def pair(j, carry):
            for b in range(2):
                i = 2 * j + b
                nb = 1 - b

                @pl.when(i >= 1)
                def _():
                    pltpu.make_async_copy(rows_b[nb], out_slice(i - 1),
                                          so_b[nb]).wait()

                @pl.when(i + 1 < _CHUNKS)
                def _():
                    pltpu.make_async_copy(idx_slice(i + 1), idx_b[nb],
                                          si_b[nb]).wait()
                    pltpu.async_copy(table_sh.at[idx_b[nb]], rows_b[nb],
                                     sg_b[nb])

                pltpu.make_async_copy(table_sh.at[idx_b[b]], rows_b[b],
                                      sg_b[b]).wait()

                @pl.when(i + 2 < _CHUNKS)
                def _():
                    pltpu.async_copy(idx_slice(i + 2), idx_b[b], si_b[b])

                pltpu.async_copy(rows_b[b], out_slice(i), so_b[b])
            return carry

        lax.fori_loop(0, _CHUNKS // 2, pair, 0)

        pltpu.make_async_copy(rows_b[1], out_slice(_CHUNKS - 1),
                              so_b[1]).wait()

    return gather_kernel


_gather = _build_gather()


def _tc_body(idx_ref, ce_ref, out_ref):
    idxs = idx_ref[0, 0, :]
    oh = (idxs[:, None] == lax.broadcasted_iota(
        jnp.int32, (_TB, MAX_LEN), 1)).astype(jnp.float32)
    out_ref[...] = jnp.dot(oh, ce_ref[...],
                           preferred_element_type=jnp.float32)


_tc_gather = pl.pallas_call(
    _tc_body,
    grid=(_NB,),
    in_specs=[
        pl.BlockSpec((1, 1, _TB), lambda i: (i, 0, 0)),
        pl.BlockSpec((MAX_LEN, N_FILTERS), lambda i: (0, 0)),
    ],
    out_specs=pl.BlockSpec((_TB, N_FILTERS), lambda i: (i, 0)),
    out_shape=jax.ShapeDtypeStruct((_B2, N_FILTERS), jnp.float32),
)


@jax.jit
def kernel(categories, ce):
    idx = categories.reshape(_B)
    out_sc = _gather(ce, idx[:_B1])
    out_tc = _tc_gather(idx[_B1:].reshape(_NB, 1, _TB), ce)
    return jnp.concatenate([out_sc, out_tc], axis=0).reshape(
        BATCH, SEQ, N_FILTERS)


# speedup vs baseline: 4.0390x; 4.0390x over previous
import functools

import jax
import jax.numpy as jnp
from jax import lax
from jax.experimental import pallas as pl
from jax.experimental.pallas import tpu as pltpu
from jax.experimental.pallas import tpu_sc as plsc

MAX_LEN = 200
N_FILTERS = 128
BATCH = 4096
SEQ = 200

_B = BATCH * SEQ
_NC = 2
_NS = 16
_NW = _NC * _NS
_PER_W = _B // _NW
_R = 400
_CHUNKS = _PER_W // _R


def _build_gather():
    mesh = plsc.VectorSubcoreMesh(core_axis_name="c", subcore_axis_name="s")

    @functools.partial(
        pl.kernel,
        mesh=mesh,
        out_type=jax.ShapeDtypeStruct((_B, N_FILTERS), jnp.float32),
        scratch_types=[
            pltpu.VMEM_SHARED((MAX_LEN, N_FILTERS), jnp.float32),
            pltpu.VMEM((_PER_W,), jnp.int32),
            pltpu.VMEM((_R, N_FILTERS), jnp.float32),
            pltpu.VMEM((_R, N_FILTERS), jnp.float32),
            pltpu.SemaphoreType.DMA,
            pltpu.SemaphoreType.DMA,
            pltpu.SemaphoreType.DMA,
            pltpu.SemaphoreType.DMA,
        ],
    )
    def gather_kernel(table_hbm, idx_hbm, out_hbm,
                      table_sh, idx_v, rows0, rows1,
                      sg0, sg1, so0, so1):
        wid = lax.axis_index("s") * _NC + lax.axis_index("c")
        base0 = wid * _PER_W
        rows_b = (rows0, rows1)
        sg_b = (sg0, sg1)
        so_b = (so0, so1)

        def out_slice(i):
            return out_hbm.at[pl.ds(base0 + i * _R, _R)]

        def idx_slice(i):
            return idx_v.at[pl.ds(i * _R, _R)]

        @pl.when(lax.axis_index("s") == 0)
        def _():
            pltpu.sync_copy(table_hbm, table_sh)

        pltpu.sync_copy(idx_hbm.at[pl.ds(base0, _PER_W)], idx_v)
        plsc.subcore_barrier()

        pltpu.async_copy(table_sh.at[idx_slice(0)], rows0, sg0)

        def pair(j, carry):
            for b in range(2):
                i = 2 * j + b
                nb = 1 - b

                @pl.when(i >= 1)
                def _():
                    pltpu.make_async_copy(rows_b[nb], out_slice(i - 1),
                                          so_b[nb]).wait()

                @pl.when(i + 1 < _CHUNKS)
                def _():
                    pltpu.async_copy(table_sh.at[idx_slice(i + 1)],
                                     rows_b[nb], sg_b[nb])

                pltpu.make_async_copy(table_sh.at[idx_slice(i)], rows_b[b],
                                      sg_b[b]).wait()

                pltpu.async_copy(rows_b[b], out_slice(i), so_b[b])
            return carry

        lax.fori_loop(0, _CHUNKS // 2, pair, 0)

        pltpu.make_async_copy(rows_b[1], out_slice(_CHUNKS - 1),
                              so_b[1]).wait()

    return gather_kernel


_gather = _build_gather()


@jax.jit
def kernel(categories, ce):
    idx = categories.reshape(_B)
    out = _gather(ce, idx)
    return out.reshape(BATCH, SEQ, N_FILTERS)


# speedup vs baseline: 4.0662x; 1.0067x over previous
import functools

import jax
import jax.numpy as jnp
from jax import lax
from jax.experimental import pallas as pl
from jax.experimental.pallas import tpu as pltpu
from jax.experimental.pallas import tpu_sc as plsc

MAX_LEN = 200
N_FILTERS = 128
BATCH = 4096
SEQ = 200

_B = BATCH * SEQ
_NC = 2
_NS = 16
_NW = _NC * _NS
_PER_W = _B // _NW
_R = 400
_CHUNKS = _PER_W // _R


def _build_gather():
    mesh = plsc.VectorSubcoreMesh(core_axis_name="c", subcore_axis_name="s")

    @functools.partial(
        pl.kernel,
        mesh=mesh,
        out_type=jax.ShapeDtypeStruct((_B, N_FILTERS), jnp.float32),
        scratch_types=[
            pltpu.VMEM_SHARED((MAX_LEN, N_FILTERS), jnp.float32),
            pltpu.VMEM((_R,), jnp.int32),
            pltpu.VMEM((_R,), jnp.int32),
            pltpu.VMEM((_R, N_FILTERS), jnp.float32),
            pltpu.VMEM((_R, N_FILTERS), jnp.float32),
            pltpu.SemaphoreType.DMA,
            pltpu.SemaphoreType.DMA,
            pltpu.SemaphoreType.DMA,
            pltpu.SemaphoreType.DMA,
            pltpu.SemaphoreType.DMA,
            pltpu.SemaphoreType.DMA,
        ],
    )
    def gather_kernel(table_hbm, idx_hbm, out_hbm,
                      table_sh, idx0, idx1, rows0, rows1,
                      si0, si1, sg0, sg1, so0, so1):
        wid = lax.axis_index("s") * _NC + lax.axis_index("c")
        base0 = wid * _PER_W
        idx_b = (idx0, idx1)
        rows_b = (rows0, rows1)
        si_b = (si0, si1)
        sg_b = (sg0, sg1)
        so_b = (so0, so1)

        def idx_slice(i):
            return idx_hbm.at[pl.ds(base0 + i * _R, _R)]

        def out_slice(i):
            return out_hbm.at[pl.ds(base0 + i * _R, _R)]

        @pl.when(lax.axis_index("s") == 0)
        def _():
            pltpu.sync_copy(table_hbm, table_sh)

        plsc.subcore_barrier()

        pltpu.async_copy(idx_slice(0), idx0, si0)
        pltpu.async_copy(idx_slice(1), idx1, si1)
        pltpu.make_async_copy(idx_slice(0), idx0, si0).wait()
        pltpu.async_copy(table_sh.at[idx0], rows0, sg0)

        def pair(j, carry):
            for b in range(2):
                i = 2 * j + b
                nb = 1 - b

                @pl.when(i >= 1)
                def _():
                    pltpu.make_async_copy(rows_b[nb], out_slice(i - 1),
                                          so_b[nb]).wait()

                @pl.when(i + 1 < _CHUNKS)
                def _():
                    pltpu.make_async_copy(idx_slice(i + 1), idx_b[nb],
                                          si_b[nb]).wait()
                    pltpu.async_copy(table_sh.at[idx_b[nb]], rows_b[nb],
                                     sg_b[nb])

                pltpu.make_async_copy(table_sh.at[idx_b[b]], rows_b[b],
                                      sg_b[b]).wait()

                @pl.when(i + 2 < _CHUNKS)
                def _():
                    pltpu.async_copy(idx_slice(i + 2), idx_b[b], si_b[b])

                pltpu.async_copy(rows_b[b], out_slice(i), so_b[b])
            return carry

        lax.fori_loop(0, _CHUNKS // 2, pair, 0)

        pltpu.make_async_copy(rows_b[1], out_slice(_CHUNKS - 1),
                              so_b[1]).wait()

    return gather_kernel


_gather = _build_gather()


@jax.jit
def kernel(categories, ce):
    idx = categories.reshape(_B)
    out = _gather(ce, idx)
    return out.reshape(BATCH, SEQ, N_FILTERS)


# speedup vs baseline: 4.1056x; 1.0097x over previous
import functools

import jax
import jax.numpy as jnp
from jax import lax
from jax.experimental import pallas as pl
from jax.experimental.pallas import tpu as pltpu
from jax.experimental.pallas import tpu_sc as plsc

MAX_LEN = 200
N_FILTERS = 128
BATCH = 4096
SEQ = 200

_B = BATCH * SEQ
_NC = 2
_NS = 16
_NW = _NC * _NS
_PER_W = _B // _NW
_R = 320
_CHUNKS = _PER_W // _R


def _build_gather():
    mesh = plsc.VectorSubcoreMesh(core_axis_name="c", subcore_axis_name="s")

    @functools.partial(
        pl.kernel,
        mesh=mesh,
        out_type=jax.ShapeDtypeStruct((_B, N_FILTERS), jnp.float32),
        scratch_types=[
            pltpu.VMEM_SHARED((MAX_LEN, N_FILTERS), jnp.float32),
            pltpu.VMEM((_R,), jnp.int32),
            pltpu.VMEM((_R,), jnp.int32),
            pltpu.VMEM((_R, N_FILTERS), jnp.float32),
            pltpu.VMEM((_R, N_FILTERS), jnp.float32),
            pltpu.SemaphoreType.DMA,
            pltpu.SemaphoreType.DMA,
            pltpu.SemaphoreType.DMA,
            pltpu.SemaphoreType.DMA,
            pltpu.SemaphoreType.DMA,
            pltpu.SemaphoreType.DMA,
        ],
    )
    def gather_kernel(table_hbm, idx_hbm, out_hbm,
                      table_sh, idx0, idx1, rows0, rows1,
                      si0, si1, sg0, sg1, so0, so1):
        wid = lax.axis_index("s") * _NC + lax.axis_index("c")
        base0 = wid * _PER_W
        idx_b = (idx0, idx1)
        rows_b = (rows0, rows1)
        si_b = (si0, si1)
        sg_b = (sg0, sg1)
        so_b = (so0, so1)

        def idx_slice(i):
            return idx_hbm.at[pl.ds(base0 + i * _R, _R)]

        def out_slice(i):
            return out_hbm.at[pl.ds(base0 + i * _R, _R)]

        @pl.when(lax.axis_index("s") == 0)
        def _():
            pltpu.sync_copy(table_hbm, table_sh)

        plsc.subcore_barrier()

        pltpu.async_copy(idx_slice(0), idx0, si0)
        pltpu.async_copy(idx_slice(1), idx1, si1)
        pltpu.make_async_copy(idx_slice(0), idx0, si0).wait()
        pltpu.async_copy(table_sh.at[idx0], rows0, sg0)

        def pair(j, carry):
            for b in range(2):
                i = 2 * j + b
                nb = 1 - b

                @pl.when(i >= 1)
                def _():
                    pltpu.make_async_copy(rows_b[nb], out_slice(i - 1),
                                          so_b[nb]).wait()

                @pl.when(i + 1 < _CHUNKS)
                def _():
                    pltpu.make_async_copy(idx_slice(i + 1), idx_b[nb],
                                          si_b[nb]).wait()
                    pltpu.async_copy(table_sh.at[idx_b[nb]], rows_b[nb],
                                     sg_b[nb])

                pltpu.make_async_copy(table_sh.at[idx_b[b]], rows_b[b],
                                      sg_b[b]).wait()

                @pl.when(i + 2 < _CHUNKS)
                def _():
                    pltpu.async_copy(idx_slice(i + 2), idx_b[b], si_b[b])

                pltpu.async_copy(rows_b[b], out_slice(i), so_b[b])
            return carry

        lax.fori_loop(0, _CHUNKS // 2, pair, 0)

        pltpu.make_async_copy(rows_b[1], out_slice(_CHUNKS - 1),
                              so_b[1]).wait()

    return gather_kernel


_gather = _build_gather()


@jax.jit
def kernel(categories, ce):
    idx = categories.reshape(_B)
    out = _gather(ce, idx)
    return out.reshape(BATCH, SEQ, N_FILTERS)


# speedup vs baseline: 4.1270x; 1.0052x over previous
import functools

import jax
import jax.numpy as jnp
from jax import lax
from jax.experimental import pallas as pl
from jax.experimental.pallas import tpu as pltpu
from jax.experimental.pallas import tpu_sc as plsc

MAX_LEN = 200
N_FILTERS = 128
BATCH = 4096
SEQ = 200

_B = BATCH * SEQ
_NC = 2
_NS = 16
_NW = _NC * _NS
_PER_W = _B // _NW
_R = 256
_CHUNKS = _PER_W // _R


def _build_gather():
    mesh = plsc.VectorSubcoreMesh(core_axis_name="c", subcore_axis_name="s")

    @functools.partial(
        pl.kernel,
        mesh=mesh,
        out_type=jax.ShapeDtypeStruct((_B, N_FILTERS), jnp.float32),
        scratch_types=[
            pltpu.VMEM_SHARED((MAX_LEN, N_FILTERS), jnp.float32),
            pltpu.VMEM((_R,), jnp.int32),
            pltpu.VMEM((_R,), jnp.int32),
            pltpu.VMEM((_R, N_FILTERS), jnp.float32),
            pltpu.VMEM((_R, N_FILTERS), jnp.float32),
            pltpu.SemaphoreType.DMA,
            pltpu.SemaphoreType.DMA,
            pltpu.SemaphoreType.DMA,
            pltpu.SemaphoreType.DMA,
            pltpu.SemaphoreType.DMA,
            pltpu.SemaphoreType.DMA,
        ],
    )
    def gather_kernel(table_hbm, idx_hbm, out_hbm,
                      table_sh, idx0, idx1, rows0, rows1,
                      si0, si1, sg0, sg1, so0, so1):
        wid = lax.axis_index("s") * _NC + lax.axis_index("c")
        base0 = wid * _PER_W
        idx_b = (idx0, idx1)
        rows_b = (rows0, rows1)
        si_b = (si0, si1)
        sg_b = (sg0, sg1)
        so_b = (so0, so1)

        def idx_slice(i):
            return idx_hbm.at[pl.ds(base0 + i * _R, _R)]

        def out_slice(i):
            return out_hbm.at[pl.ds(base0 + i * _R, _R)]

        @pl.when(lax.axis_index("s") == 0)
        def _():
            pltpu.sync_copy(table_hbm, table_sh)

        plsc.subcore_barrier()

        pltpu.async_copy(idx_slice(0), idx0, si0)
        pltpu.async_copy(idx_slice(1), idx1, si1)
        pltpu.make_async_copy(idx_slice(0), idx0, si0).wait()
        pltpu.async_copy(table_sh.at[idx0], rows0, sg0)

        def pair(j, carry):
            for b in range(2):
                i = 2 * j + b
                nb = 1 - b

                @pl.when(i >= 1)
                def _():
                    pltpu.make_async_copy(rows_b[nb], out_slice(i - 1),
                                          so_b[nb]).wait()

                @pl.when(i + 1 < _CHUNKS)
                def _():
                    pltpu.make_async_copy(idx_slice(i + 1), idx_b[nb],
                                          si_b[nb]).wait()
                    pltpu.async_copy(table_sh.at[idx_b[nb]], rows_b[nb],
                                     sg_b[nb])

                pltpu.make_async_copy(table_sh.at[idx_b[b]], rows_b[b],
                                      sg_b[b]).wait()

                @pl.when(i + 2 < _CHUNKS)
                def _():
                    pltpu.async_copy(idx_slice(i + 2), idx_b[b], si_b[b])

                pltpu.async_copy(rows_b[b], out_slice(i), so_b[b])
            return carry

        lax.fori_loop(0, _CHUNKS // 2, pair, 0)

        pltpu.make_async_copy(rows_b[1], out_slice(_CHUNKS - 1),
                              so_b[1]).wait()

    return gather_kernel


_gather = _build_gather()


@jax.jit
def kernel(categories, ce):
    idx = categories.reshape(_B)
    out = _gather(ce, idx)
    return out.reshape(BATCH, SEQ, N_FILTERS)


# speedup vs baseline: 4.1330x; 1.0014x over previous
import functools

import jax
import jax.numpy as jnp
from jax import lax
from jax.experimental import pallas as pl
from jax.experimental.pallas import tpu as pltpu
from jax.experimental.pallas import tpu_sc as plsc

MAX_LEN = 200
N_FILTERS = 128
BATCH = 4096
SEQ = 200

_B = BATCH * SEQ
_NC = 2
_NS = 16
_NW = _NC * _NS
_PER_W = _B // _NW
_R = 160
_CHUNKS = _PER_W // _R


def _build_gather():
    mesh = plsc.VectorSubcoreMesh(core_axis_name="c", subcore_axis_name="s")

    @functools.partial(
        pl.kernel,
        mesh=mesh,
        out_type=jax.ShapeDtypeStruct((_B, N_FILTERS), jnp.float32),
        scratch_types=[
            pltpu.VMEM_SHARED((MAX_LEN, N_FILTERS), jnp.float32),
            pltpu.VMEM((_R,), jnp.int32),
            pltpu.VMEM((_R,), jnp.int32),
            pltpu.VMEM((_R, N_FILTERS), jnp.float32),
            pltpu.VMEM((_R, N_FILTERS), jnp.float32),
            pltpu.SemaphoreType.DMA,
            pltpu.SemaphoreType.DMA,
            pltpu.SemaphoreType.DMA,
            pltpu.SemaphoreType.DMA,
            pltpu.SemaphoreType.DMA,
            pltpu.SemaphoreType.DMA,
        ],
    )
    def gather_kernel(table_hbm, idx_hbm, out_hbm,
                      table_sh, idx0, idx1, rows0, rows1,
                      si0, si1, sg0, sg1, so0, so1):
        wid = lax.axis_index("s") * _NC + lax.axis_index("c")
        base0 = wid * _PER_W
        idx_b = (idx0, idx1)
        rows_b = (rows0, rows1)
        si_b = (si0, si1)
        sg_b = (sg0, sg1)
        so_b = (so0, so1)

        def idx_slice(i):
            return idx_hbm.at[pl.ds(base0 + i * _R, _R)]

        def out_slice(i):
            return out_hbm.at[pl.ds(base0 + i * _R, _R)]

        @pl.when(lax.axis_index("s") == 0)
        def _():
            pltpu.sync_copy(table_hbm, table_sh)

        plsc.subcore_barrier()

        pltpu.async_copy(idx_slice(0), idx0, si0)
        pltpu.async_copy(idx_slice(1), idx1, si1)
        pltpu.make_async_copy(idx_slice(0), idx0, si0).wait()
        pltpu.async_copy(table_sh.at[idx0], rows0, sg0)

        def pair(j, carry):
            for b in range(2):
                i = 2 * j + b
                nb = 1 - b

                @pl.when(i >= 1)
                def _():
                    pltpu.make_async_copy(rows_b[nb], out_slice(i - 1),
                                          so_b[nb]).wait()

                @pl.when(i + 1 < _CHUNKS)
                def _():
                    pltpu.make_async_copy(idx_slice(i + 1), idx_b[nb],
                                          si_b[nb]).wait()
                    pltpu.async_copy(table_sh.at[idx_b[nb]], rows_b[nb],
                                     sg_b[nb])

                pltpu.make_async_copy(table_sh.at[idx_b[b]], rows_b[b],
                                      sg_b[b]).wait()

                @pl.when(i + 2 < _CHUNKS)
                def _():
                    pltpu.async_copy(idx_slice(i + 2), idx_b[b], si_b[b])

                pltpu.async_copy(rows_b[b], out_slice(i), so_b[b])
            return carry

        lax.fori_loop(0, _CHUNKS // 2, pair, 0)

        pltpu.make_async_copy(rows_b[1], out_slice(_CHUNKS - 1),
                              so_b[1]).wait()

    return gather_kernel


_gather = _build_gather()


@jax.jit
def kernel(categories, ce):
    idx = categories.reshape(_B)
    out = _gather(ce, idx)
    return out.reshape(BATCH, SEQ, N_FILTERS)


# speedup vs baseline: 4.1365x; 1.0009x over previous
import functools

import jax
import jax.numpy as jnp
from jax import lax
from jax.experimental import pallas as pl
from jax.experimental.pallas import tpu as pltpu
from jax.experimental.pallas import tpu_sc as plsc

MAX_LEN = 200
N_FILTERS = 128
BATCH = 4096
SEQ = 200

_B = BATCH * SEQ
_NC = 2
_NS = 16
_NW = _NC * _NS
_PER_W = _B // _NW
_R = 256
_CHUNKS = _PER_W // _R


def _build_gather():
    mesh = plsc.VectorSubcoreMesh(core_axis_name="c", subcore_axis_name="s")

    @functools.partial(
        pl.kernel,
        mesh=mesh,
        out_type=jax.ShapeDtypeStruct((_B, N_FILTERS), jnp.float32),
        scratch_types=[
            pltpu.VMEM_SHARED((MAX_LEN, N_FILTERS), jnp.float32),
            pltpu.VMEM((_R,), jnp.int32),
            pltpu.VMEM((_R,), jnp.int32),
            pltpu.VMEM((_R, N_FILTERS), jnp.float32),
            pltpu.VMEM((_R, N_FILTERS), jnp.float32),
            pltpu.SemaphoreType.DMA,
            pltpu.SemaphoreType.DMA,
            pltpu.SemaphoreType.DMA,
            pltpu.SemaphoreType.DMA,
            pltpu.SemaphoreType.DMA,
            pltpu.SemaphoreType.DMA,
        ],
    )
    def gather_kernel(table_hbm, idx_hbm, out_hbm,
                      table_sh, idx0, idx1, rows0, rows1,
                      si0, si1, sg0, sg1, so0, so1):
        wid = lax.axis_index("s") * _NC + lax.axis_index("c")
        base0 = wid * _PER_W
        idx_b = (idx0, idx1)
        rows_b = (rows0, rows1)
        si_b = (si0, si1)
        sg_b = (sg0, sg1)
        so_b = (so0, so1)

        def idx_slice(i):
            return idx_hbm.at[pl.ds(base0 + i * _R, _R)]

        def out_slice(i):
            return out_hbm.at[pl.ds(base0 + i * _R, _R)]

        @pl.when(lax.axis_index("s") == 0)
        def _():
            pltpu.sync_copy(table_hbm, table_sh)

        plsc.subcore_barrier()

        pltpu.async_copy(idx_slice(0), idx0, si0)
        pltpu.async_copy(idx_slice(1), idx1, si1)
        pltpu.make_async_copy(idx_slice(0), idx0, si0).wait()
        pltpu.async_copy(table_sh.at[idx0], rows0, sg0)

        def pair(j, carry):
            for b in range(2):
                i = 2 * j + b
                nb = 1 - b

                @pl.when(i >= 1)
                def _():
                    pltpu.make_async_copy(rows_b[nb], out_slice(i - 1),
                                          so_b[nb]).wait()

                @pl.when(i + 1 < _CHUNKS)
                def _():
                    pltpu.make_async_copy(idx_slice(i + 1), idx_b[nb],
                                          si_b[nb]).wait()
                    pltpu.async_copy(table_sh.at[idx_b[nb]], rows_b[nb],
                                     sg_b[nb])

                pltpu.make_async_copy(table_sh.at[idx_b[b]], rows_b[b],
                                      sg_b[b]).wait()

                @pl.when(i + 2 < _CHUNKS)
                def _():
                    pltpu.async_copy(idx_slice(i + 2), idx_b[b], si_b[b])

                pltpu.async_copy(rows_b[b], out_slice(i), so_b[b])
            return carry

        lax.fori_loop(0, _CHUNKS // 2, pair, 0)

        pltpu.make_async_copy(rows_b[1], out_slice(_CHUNKS - 1),
                              so_b[1]).wait()

    return gather_kernel


_gather = _build_gather()


@jax.jit
def kernel(categories, ce):
    idx = categories.reshape(_B)
    out = _gather(ce, idx)
    return out.reshape(BATCH, SEQ, N_FILTERS)
